# baseline (speedup 1.0000x reference)
import functools

import jax
import jax.numpy as jnp
from jax import lax
from jax.experimental import pallas as pl
from jax.experimental.pallas import tpu as pltpu
from jax.experimental.pallas import tpu_sc as plsc

TOPK = 4000
NHEADS = 8
DHEAD = 128


def _mm_kernel(a_ref, b_ref, o_ref):
    o_ref[...] = jnp.dot(
        a_ref[...], b_ref[...], preferred_element_type=jnp.float32
    ).astype(o_ref.dtype)


def _mm_bias_kernel(a_ref, b_ref, bias_ref, o_ref):
    o_ref[...] = (
        jnp.dot(a_ref[...], b_ref[...], preferred_element_type=jnp.float32)
        + bias_ref[...]
    )


def _matmul(a, b, blk=512, out_dtype=jnp.float32):
    m, k = a.shape
    n = b.shape[1]
    return pl.pallas_call(
        _mm_kernel,
        grid=(m // blk,),
        in_specs=[
            pl.BlockSpec((blk, k), lambda i: (i, 0)),
            pl.BlockSpec((k, n), lambda i: (0, 0)),
        ],
        out_specs=pl.BlockSpec((blk, n), lambda i: (i, 0)),
        out_shape=jax.ShapeDtypeStruct((m, n), out_dtype),
    )(a, b)


def _matmul_bias(a, b, bias, blk=512):
    m, k = a.shape
    n = b.shape[1]
    return pl.pallas_call(
        _mm_bias_kernel,
        grid=(m // blk,),
        in_specs=[
            pl.BlockSpec((blk, k), lambda i: (i, 0)),
            pl.BlockSpec((k, n), lambda i: (0, 0)),
            pl.BlockSpec((1, n), lambda i: (0, 0)),
        ],
        out_specs=pl.BlockSpec((blk, n), lambda i: (i, 0)),
        out_shape=jax.ShapeDtypeStruct((m, n), jnp.float32),
    )(a, b, bias.reshape(1, n))


def _conv_kernel(he_ref, ho_ref, hes_ref, w0_ref, w1_ref, w2_ref, b_ref, o_ref):
    acc = jnp.dot(he_ref[...], w0_ref[...], preferred_element_type=jnp.float32)
    acc += jnp.dot(ho_ref[...], w1_ref[...], preferred_element_type=jnp.float32)
    acc += jnp.dot(hes_ref[...], w2_ref[...], preferred_element_type=jnp.float32)
    o_ref[...] = jax.nn.gelu(acc + b_ref[...])


def _conv_block(h, w, b):
    bsz, lin, c = h.shape
    lout = lin // 2
    he = h[:, 0::2, :]
    ho = h[:, 1::2, :]
    hes = jnp.concatenate([he[:, 1:, :], jnp.zeros((bsz, 1, c), jnp.float32)], axis=1)
    rows = bsz * lout
    blk = min(512, rows)
    out = pl.pallas_call(
        _conv_kernel,
        grid=(rows // blk,),
        in_specs=[
            pl.BlockSpec((blk, c), lambda i: (i, 0)),
            pl.BlockSpec((blk, c), lambda i: (i, 0)),
            pl.BlockSpec((blk, c), lambda i: (i, 0)),
            pl.BlockSpec((c, c), lambda i: (0, 0)),
            pl.BlockSpec((c, c), lambda i: (0, 0)),
            pl.BlockSpec((c, c), lambda i: (0, 0)),
            pl.BlockSpec((1, c), lambda i: (0, 0)),
        ],
        out_specs=pl.BlockSpec((blk, c), lambda i: (i, 0)),
        out_shape=jax.ShapeDtypeStruct((rows, c), jnp.float32),
    )(
        he.reshape(rows, c),
        ho.reshape(rows, c),
        hes.reshape(rows, c),
        w[0],
        w[1],
        w[2],
        b.reshape(1, c),
    )
    return out.reshape(bsz, lout, c)


def _rank_kernel(s_ref, st_ref, ones_ref, out_ref):
    bsz, n = s_ref.shape
    jblk = 512
    for b in range(bsz):
        scol = st_ref[:, b : b + 1]
        acc = jnp.zeros((n, 128), jnp.float32)
        for jb in range(n // jblk):
            srow = s_ref[b : b + 1, jb * jblk : (jb + 1) * jblk]
            jj = lax.broadcasted_iota(jnp.int32, (n, jblk), 1) + jb * jblk
            ii = lax.broadcasted_iota(jnp.int32, (n, jblk), 0)
            m = ((srow > scol) | ((srow == scol) & (jj < ii))).astype(jnp.float32)
            acc += jnp.dot(m, ones_ref[...], preferred_element_type=jnp.float32)
        out_ref[:, b : b + 1] = acc[:, 0:1].astype(jnp.int32) + b * n


def _rank_full(s):
    bsz, n = s.shape
    return pl.pallas_call(
        _rank_kernel,
        in_specs=[
            pl.BlockSpec((bsz, n), lambda: (0, 0)),
            pl.BlockSpec((n, bsz), lambda: (0, 0)),
            pl.BlockSpec((512, 128), lambda: (0, 0)),
        ],
        out_specs=pl.BlockSpec((n, bsz), lambda: (0, 0)),
        out_shape=jax.ShapeDtypeStruct((n, bsz), jnp.int32),
    )(s, s.T, jnp.ones((512, 128), jnp.float32))


def _attn_kernel(q_ref, k_ref, v_ref, wf_ref, bp_ref, o_ref):
    q = q_ref[...]
    k = k_ref[0]
    v = v_ref[0]
    scale = 1.0 / (DHEAD ** 0.5)
    outs = []
    for h in range(NHEADS):
        sl = slice(h * DHEAD, (h + 1) * DHEAD)
        qh = q[:, sl]
        kh = k[:, sl]
        vh = v[:, sl]
        scores = (
            lax.dot_general(qh, kh, (((1,), (1,)), ((), ())),
                            preferred_element_type=jnp.float32)
            * scale
        )
        m = jnp.max(scores, axis=-1, keepdims=True)
        p = jnp.exp(scores - m)
        p = p / jnp.sum(p, axis=-1, keepdims=True)
        outs.append(
            jnp.dot(p.astype(vh.dtype), vh, preferred_element_type=jnp.float32)
        )
    o = jnp.concatenate(outs, axis=1)
    o_ref[...] = (
        jnp.dot(o.astype(wf_ref.dtype), wf_ref[...],
                preferred_element_type=jnp.float32)
        + bp_ref[...]
    )[None]


def _attention(qpre, k3, v3, wfold, b_pool, mblk=512):
    bsz, lc, c = k3.shape
    m = qpre.shape[0]
    return pl.pallas_call(
        _attn_kernel,
        grid=(bsz, m // mblk),
        in_specs=[
            pl.BlockSpec((mblk, c), lambda b, i: (i, 0)),
            pl.BlockSpec((1, lc, c), lambda b, i: (b, 0, 0)),
            pl.BlockSpec((1, lc, c), lambda b, i: (b, 0, 0)),
            pl.BlockSpec((c, c), lambda b, i: (0, 0)),
            pl.BlockSpec((1, c), lambda b, i: (0, 0)),
        ],
        out_specs=pl.BlockSpec((1, mblk, c), lambda b, i: (b, i, 0)),
        out_shape=jax.ShapeDtypeStruct((bsz, m, c), jnp.float32),
    )(qpre, k3, v3, wfold, b_pool.reshape(1, c))


def _kv_kernel(cf_ref, wk_ref, wv_ref, k_ref, v_ref):
    cf = cf_ref[0]
    k_ref[...] = jnp.dot(cf, wk_ref[...], preferred_element_type=jnp.float32)[
        None
    ].astype(k_ref.dtype)
    v_ref[...] = jnp.dot(cf, wv_ref[...], preferred_element_type=jnp.float32)[
        None
    ].astype(v_ref.dtype)


def _kv(conv_feats, wk, wv):
    bsz, lc, c = conv_feats.shape
    return pl.pallas_call(
        _kv_kernel,
        grid=(bsz,),
        in_specs=[
            pl.BlockSpec((1, lc, c), lambda b: (b, 0, 0)),
            pl.BlockSpec((c, c), lambda b: (0, 0)),
            pl.BlockSpec((c, c), lambda b: (0, 0)),
        ],
        out_specs=[
            pl.BlockSpec((1, lc, c), lambda b: (b, 0, 0)),
            pl.BlockSpec((1, lc, c), lambda b: (b, 0, 0)),
        ],
        out_shape=[
            jax.ShapeDtypeStruct((bsz, lc, c), conv_feats.dtype),
            jax.ShapeDtypeStruct((bsz, lc, c), conv_feats.dtype),
        ],
    )(conv_feats, wk, wv)


def _sc_scatter(table, rank_all, vals):
    r, d = table.shape
    info = plsc.get_sparse_core_info()
    nw = info.num_cores * info.num_subcores
    per_w = r // nw
    ch = 64
    n_ch = per_w // ch
    nc = info.num_cores
    mesh = plsc.VectorSubcoreMesh(core_axis_name="c", subcore_axis_name="s")

    @functools.partial(
        pl.kernel,
        out_type=[
            jax.ShapeDtypeStruct((r, d), jnp.float32),
            jax.ShapeDtypeStruct((r,), jnp.int32),
        ],
        mesh=mesh,
        scratch_types=[
            pltpu.VMEM((ch,), jnp.int32),
            pltpu.VMEM((ch,), jnp.int32),
            pltpu.VMEM((ch, d), jnp.float32),
            pltpu.SemaphoreType.DMA,
        ],
    )
    def sk(table_hbm, rank_hbm, vals_hbm, rows_out, idx_out, rank_v, vals_v,
           rows_v, sem):
        wid = lax.axis_index("s") * nc + lax.axis_index("c")
        base = wid * per_w
        for c in range(n_ch):
            off = base + c * ch
            pltpu.sync_copy(rank_hbm.at[pl.ds(off, ch)], rank_v)
            pltpu.sync_copy(table_hbm.at[pl.ds(off, ch)], rows_v)
            pltpu.async_copy(rows_v, rows_out.at[rank_v], sem).wait()
            pltpu.sync_copy(vals_hbm.at[pl.ds(off, ch)], vals_v)
            pltpu.async_copy(vals_v, idx_out.at[rank_v], sem).wait()

    return sk(table, rank_all, vals)


def kernel(embeddings, W_proj, b_proj, conv_w0, conv_b0, conv_w1, conv_b1,
           conv_w2, conv_b2, queries, Wq, Wk, Wv, Wo, w_score, b_score,
           W_pool, b_pool, W_pass, b_pass):
    bsz, l, d = embeddings.shape

    x = embeddings @ W_proj + b_proj
    h0 = lax.conv_general_dilated(
        x, conv_w0, window_strides=(2,), padding="SAME",
        dimension_numbers=("NWC", "WIO", "NWC"),
    ) + conv_b0
    h0 = jax.nn.gelu(h0)
    s = x @ w_score + b_score

    h1 = _conv_block(h0, conv_w1, conv_b1)
    conv_feats = _conv_block(h1, conv_w2, conv_b2)

    rank_nb = _rank_full(s)
    rank_all = rank_nb.T.reshape(bsz * l)
    vals = jnp.tile(jnp.arange(l, dtype=jnp.int32), bsz)

    pass_full = _matmul_bias(embeddings.reshape(bsz * l, d), W_pass, b_pass)
    rows_out, idx_all = _sc_scatter(pass_full, rank_all, vals)
    idx = idx_all.reshape(bsz, l)[:, :TOPK]
    t_pass = rows_out.reshape(bsz, l, -1)[:, :TOPK, :]

    bf = jnp.bfloat16
    qpre = _matmul(queries.astype(bf), Wq.astype(bf), out_dtype=bf)
    k3, v3 = _kv(conv_feats.astype(bf), Wk.astype(bf), Wv.astype(bf))
    wfold = _matmul(Wo, W_pool)
    t_pool = _attention(qpre, k3, v3, wfold.astype(bf), b_pool)

    transformer_input = jnp.concatenate([t_pool, t_pass], axis=1)
    return (transformer_input, idx, conv_feats)


# speedup vs baseline: 1.1302x; 1.1302x over previous
import functools

import jax
import jax.numpy as jnp
from jax import lax
from jax.experimental import pallas as pl
from jax.experimental.pallas import tpu as pltpu
from jax.experimental.pallas import tpu_sc as plsc

TOPK = 4000
NHEADS = 8
DHEAD = 128


def _mm_kernel(a_ref, b_ref, o_ref):
    o_ref[...] = jnp.dot(
        a_ref[...], b_ref[...], preferred_element_type=jnp.float32
    ).astype(o_ref.dtype)


def _mm_bias_kernel(a_ref, b_ref, bias_ref, o_ref):
    o_ref[...] = (
        jnp.dot(a_ref[...], b_ref[...], preferred_element_type=jnp.float32)
        + bias_ref[...]
    )


def _matmul(a, b, blk=512, out_dtype=jnp.float32):
    m, k = a.shape
    n = b.shape[1]
    return pl.pallas_call(
        _mm_kernel,
        grid=(m // blk,),
        in_specs=[
            pl.BlockSpec((blk, k), lambda i: (i, 0)),
            pl.BlockSpec((k, n), lambda i: (0, 0)),
        ],
        out_specs=pl.BlockSpec((blk, n), lambda i: (i, 0)),
        out_shape=jax.ShapeDtypeStruct((m, n), out_dtype),
    )(a, b)


def _matmul_bias(a, b, bias, blk=512):
    m, k = a.shape
    n = b.shape[1]
    return pl.pallas_call(
        _mm_bias_kernel,
        grid=(m // blk,),
        in_specs=[
            pl.BlockSpec((blk, k), lambda i: (i, 0)),
            pl.BlockSpec((k, n), lambda i: (0, 0)),
            pl.BlockSpec((1, n), lambda i: (0, 0)),
        ],
        out_specs=pl.BlockSpec((blk, n), lambda i: (i, 0)),
        out_shape=jax.ShapeDtypeStruct((m, n), jnp.float32),
    )(a, b, bias.reshape(1, n))


def _conv_kernel(hp_ref, hs_ref, w0_ref, w1_ref, w2_ref, b_ref, o_ref):
    c = w0_ref.shape[0]
    acc = jnp.dot(hp_ref[:, :c], w0_ref[...], preferred_element_type=jnp.float32)
    acc += jnp.dot(hp_ref[:, c:], w1_ref[...], preferred_element_type=jnp.float32)
    acc += jnp.dot(hs_ref[...], w2_ref[...], preferred_element_type=jnp.float32)
    o_ref[...] = jax.nn.gelu(acc + b_ref[...])


def _conv_block(h, w, b):
    bsz, lin, c = h.shape
    lout = lin // 2
    hp = h.reshape(bsz, lout, 2 * c)
    hs = jnp.concatenate(
        [hp[:, 1:, :c], jnp.zeros((bsz, 1, c), jnp.float32)], axis=1
    )
    rows = bsz * lout
    blk = min(512, rows)
    out = pl.pallas_call(
        _conv_kernel,
        grid=(rows // blk,),
        in_specs=[
            pl.BlockSpec((blk, 2 * c), lambda i: (i, 0)),
            pl.BlockSpec((blk, c), lambda i: (i, 0)),
            pl.BlockSpec((c, c), lambda i: (0, 0)),
            pl.BlockSpec((c, c), lambda i: (0, 0)),
            pl.BlockSpec((c, c), lambda i: (0, 0)),
            pl.BlockSpec((1, c), lambda i: (0, 0)),
        ],
        out_specs=pl.BlockSpec((blk, c), lambda i: (i, 0)),
        out_shape=jax.ShapeDtypeStruct((rows, c), jnp.float32),
    )(
        hp.reshape(rows, 2 * c),
        hs.reshape(rows, c),
        w[0],
        w[1],
        w[2],
        b.reshape(1, c),
    )
    return out.reshape(bsz, lout, c)


def _rank_kernel(s_ref, st_ref, ones_ref, out_ref):
    bsz, n = s_ref.shape
    jblk = 512
    for b in range(bsz):
        scol = st_ref[:, b : b + 1]
        acc = jnp.zeros((n, 128), jnp.float32)
        for jb in range(n // jblk):
            srow = s_ref[b : b + 1, jb * jblk : (jb + 1) * jblk]
            jj = lax.broadcasted_iota(jnp.int32, (n, jblk), 1) + jb * jblk
            ii = lax.broadcasted_iota(jnp.int32, (n, jblk), 0)
            m = ((srow > scol) | ((srow == scol) & (jj < ii))).astype(jnp.float32)
            acc += jnp.dot(m, ones_ref[...], preferred_element_type=jnp.float32)
        out_ref[:, b : b + 1] = acc[:, 0:1].astype(jnp.int32) + b * n


def _rank_full(s):
    bsz, n = s.shape
    return pl.pallas_call(
        _rank_kernel,
        in_specs=[
            pl.BlockSpec((bsz, n), lambda: (0, 0)),
            pl.BlockSpec((n, bsz), lambda: (0, 0)),
            pl.BlockSpec((512, 128), lambda: (0, 0)),
        ],
        out_specs=pl.BlockSpec((n, bsz), lambda: (0, 0)),
        out_shape=jax.ShapeDtypeStruct((n, bsz), jnp.int32),
    )(s, s.T, jnp.ones((512, 128), jnp.float32))


def _attn_kernel(q_ref, k_ref, v_ref, wf_ref, bp_ref, o_ref):
    q = q_ref[...]
    k = k_ref[0]
    v = v_ref[0]
    scale = 1.0 / (DHEAD ** 0.5)
    outs = []
    for h in range(NHEADS):
        sl = slice(h * DHEAD, (h + 1) * DHEAD)
        qh = q[:, sl]
        kh = k[:, sl]
        vh = v[:, sl]
        scores = (
            lax.dot_general(qh, kh, (((1,), (1,)), ((), ())),
                            preferred_element_type=jnp.float32)
            * scale
        )
        m = jnp.max(scores, axis=-1, keepdims=True)
        p = jnp.exp(scores - m)
        p = p / jnp.sum(p, axis=-1, keepdims=True)
        outs.append(
            jnp.dot(p.astype(vh.dtype), vh, preferred_element_type=jnp.float32)
        )
    o = jnp.concatenate(outs, axis=1)
    o_ref[...] = (
        jnp.dot(o.astype(wf_ref.dtype), wf_ref[...],
                preferred_element_type=jnp.float32)
        + bp_ref[...]
    )[None]


def _attention(qpre, k3, v3, wfold, b_pool, mblk=512):
    bsz, lc, c = k3.shape
    m = qpre.shape[0]
    return pl.pallas_call(
        _attn_kernel,
        grid=(bsz, m // mblk),
        in_specs=[
            pl.BlockSpec((mblk, c), lambda b, i: (i, 0)),
            pl.BlockSpec((1, lc, c), lambda b, i: (b, 0, 0)),
            pl.BlockSpec((1, lc, c), lambda b, i: (b, 0, 0)),
            pl.BlockSpec((c, c), lambda b, i: (0, 0)),
            pl.BlockSpec((1, c), lambda b, i: (0, 0)),
        ],
        out_specs=pl.BlockSpec((1, mblk, c), lambda b, i: (b, i, 0)),
        out_shape=jax.ShapeDtypeStruct((bsz, m, c), jnp.float32),
    )(qpre, k3, v3, wfold, b_pool.reshape(1, c))


def _kv_kernel(cf_ref, wk_ref, wv_ref, k_ref, v_ref):
    cf = cf_ref[0]
    k_ref[...] = jnp.dot(cf, wk_ref[...], preferred_element_type=jnp.float32)[
        None
    ].astype(k_ref.dtype)
    v_ref[...] = jnp.dot(cf, wv_ref[...], preferred_element_type=jnp.float32)[
        None
    ].astype(v_ref.dtype)


def _kv(conv_feats, wk, wv):
    bsz, lc, c = conv_feats.shape
    return pl.pallas_call(
        _kv_kernel,
        grid=(bsz,),
        in_specs=[
            pl.BlockSpec((1, lc, c), lambda b: (b, 0, 0)),
            pl.BlockSpec((c, c), lambda b: (0, 0)),
            pl.BlockSpec((c, c), lambda b: (0, 0)),
        ],
        out_specs=[
            pl.BlockSpec((1, lc, c), lambda b: (b, 0, 0)),
            pl.BlockSpec((1, lc, c), lambda b: (b, 0, 0)),
        ],
        out_shape=[
            jax.ShapeDtypeStruct((bsz, lc, c), conv_feats.dtype),
            jax.ShapeDtypeStruct((bsz, lc, c), conv_feats.dtype),
        ],
    )(conv_feats, wk, wv)


def _sc_scatter(table, rank_all, vals):
    r, d = table.shape
    info = plsc.get_sparse_core_info()
    nw = info.num_cores * info.num_subcores
    per_w = r // nw
    ch = 64
    n_ch = per_w // ch
    nc = info.num_cores
    mesh = plsc.VectorSubcoreMesh(core_axis_name="c", subcore_axis_name="s")

    @functools.partial(
        pl.kernel,
        out_type=[
            jax.ShapeDtypeStruct((r, d), jnp.float32),
            jax.ShapeDtypeStruct((r,), jnp.int32),
        ],
        mesh=mesh,
        scratch_types=[
            pltpu.VMEM((ch,), jnp.int32),
            pltpu.VMEM((ch,), jnp.int32),
            pltpu.VMEM((ch, d), jnp.float32),
            pltpu.SemaphoreType.DMA,
        ],
    )
    def sk(table_hbm, rank_hbm, vals_hbm, rows_out, idx_out, rank_v, vals_v,
           rows_v, sem):
        wid = lax.axis_index("s") * nc + lax.axis_index("c")
        base = wid * per_w
        for c in range(n_ch):
            off = base + c * ch
            pltpu.sync_copy(rank_hbm.at[pl.ds(off, ch)], rank_v)
            pltpu.sync_copy(table_hbm.at[pl.ds(off, ch)], rows_v)
            pltpu.async_copy(rows_v, rows_out.at[rank_v], sem).wait()
            pltpu.sync_copy(vals_hbm.at[pl.ds(off, ch)], vals_v)
            pltpu.async_copy(vals_v, idx_out.at[rank_v], sem).wait()

    return sk(table, rank_all, vals)


def kernel(embeddings, W_proj, b_proj, conv_w0, conv_b0, conv_w1, conv_b1,
           conv_w2, conv_b2, queries, Wq, Wk, Wv, Wo, w_score, b_score,
           W_pool, b_pool, W_pass, b_pass):
    bsz, l, d = embeddings.shape

    x = embeddings @ W_proj + b_proj
    h0 = lax.conv_general_dilated(
        x, conv_w0, window_strides=(2,), padding="SAME",
        dimension_numbers=("NWC", "WIO", "NWC"),
    ) + conv_b0
    h0 = jax.nn.gelu(h0)
    s = x @ w_score + b_score

    h1 = _conv_block(h0, conv_w1, conv_b1)
    conv_feats = _conv_block(h1, conv_w2, conv_b2)

    rank_nb = _rank_full(s)
    rank_all = rank_nb.T.reshape(bsz * l)
    vals = jnp.tile(jnp.arange(l, dtype=jnp.int32), bsz)

    pass_full = _matmul_bias(embeddings.reshape(bsz * l, d), W_pass, b_pass)
    rows_out, idx_all = _sc_scatter(pass_full, rank_all, vals)
    idx = idx_all.reshape(bsz, l)[:, :TOPK]
    t_pass = rows_out.reshape(bsz, l, -1)[:, :TOPK, :]

    qpre = _matmul(queries, Wq)
    k3, v3 = _kv(conv_feats, Wk, Wv)
    wfold = _matmul(Wo, W_pool)
    t_pool = _attention(qpre, k3, v3, wfold, b_pool)

    transformer_input = jnp.concatenate([t_pool, t_pass], axis=1)
    return (transformer_input, idx, conv_feats)
